# baseline (speedup 1.0000x reference)
import functools

import jax
import jax.numpy as jnp
from jax import lax
from jax.experimental import pallas as pl
from jax.experimental.pallas import tpu as pltpu
from jax.experimental.pallas import tpu_sc as plsc

_DEPTH = 32
_SPAN = 1024
_LANES = 16


def _gather_body(tablet_hbm, x_hbm, out_hbm, idx_v, tablet_v, vals_v, *,
                 spans_per_b):
    wid = lax.axis_index("s") * 2 + lax.axis_index("c")
    b = wid // spans_per_b
    t0 = (wid % spans_per_b) * _SPAN

    pltpu.sync_copy(tablet_hbm, tablet_v)
    pltpu.sync_copy(x_hbm.at[b, pl.ds(t0, _SPAN)], idx_v)

    row_ids = [jnp.full((_LANES,), d, jnp.int32) for d in range(_DEPTH)]

    @plsc.parallel_loop(0, _SPAN, step=_LANES, unroll=1)
    def group(i0):
        idx16 = idx_v[pl.ds(i0, _LANES)]
        for d in range(_DEPTH):
            vals_v[d, pl.ds(i0, _LANES)] = plsc.load_gather(
                tablet_v, [row_ids[d], idx16])

    pltpu.sync_copy(vals_v, out_hbm.at[b, :, pl.ds(t0, _SPAN)])


@jax.jit
def kernel(x, byte2dsn):
    b, t = x.shape
    spans_per_b = t // _SPAN
    x = x.astype(jnp.int32)
    tablet = jnp.transpose(byte2dsn)

    mesh = plsc.VectorSubcoreMesh(core_axis_name="c", subcore_axis_name="s")
    gather = pl.kernel(
        functools.partial(_gather_body, spans_per_b=spans_per_b),
        mesh=mesh,
        out_type=jax.ShapeDtypeStruct((b, _DEPTH, t), jnp.float32),
        scratch_types=[
            pltpu.VMEM((_SPAN,), jnp.int32),
            pltpu.VMEM((_DEPTH, 256), jnp.float32),
            pltpu.VMEM((_DEPTH, _SPAN), jnp.float32),
        ],
        compiler_params=pltpu.CompilerParams(
            use_tc_tiling_on_sc=True,
            needs_layout_passes=False,
            disable_bounds_checks=True,
            disable_semaphore_checks=True,
            skip_device_barrier=True,
        ),
    )
    out_t = gather(tablet, x)
    return jnp.transpose(out_t, (0, 2, 1))


# speedup vs baseline: 1.0177x; 1.0177x over previous
import functools

import jax
import jax.numpy as jnp
from jax import lax
from jax.experimental import pallas as pl
from jax.experimental.pallas import tpu as pltpu
from jax.experimental.pallas import tpu_sc as plsc

_DEPTH = 32
_SPAN = 1024
_LANES = 16


def _gather_body(tablet_hbm, x_hbm, out_hbm, idx_v, tablet_v, vals_v,
                 sem_in, *, spans_per_b):
    wid = lax.axis_index("s") * 2 + lax.axis_index("c")
    b = wid // spans_per_b
    t0 = (wid % spans_per_b) * _SPAN

    c_table = pltpu.async_copy(tablet_hbm, tablet_v, sem_in)
    c_idx = pltpu.async_copy(x_hbm.at[b, pl.ds(t0, _SPAN)], idx_v, sem_in)
    c_table.wait()
    c_idx.wait()

    row_ids = [jnp.full((_LANES,), d, jnp.int32) for d in range(_DEPTH)]

    @plsc.parallel_loop(0, _SPAN, step=_LANES, unroll=2)
    def group(i0):
        idx16 = idx_v[pl.ds(i0, _LANES)]
        for d in range(_DEPTH):
            vals_v[d, pl.ds(i0, _LANES)] = plsc.load_gather(
                tablet_v, [row_ids[d], idx16])

    pltpu.sync_copy(vals_v, out_hbm.at[b, :, pl.ds(t0, _SPAN)])


@jax.jit
def kernel(x, byte2dsn):
    b, t = x.shape
    spans_per_b = t // _SPAN
    x = x.astype(jnp.int32)
    tablet = jnp.transpose(byte2dsn)

    mesh = plsc.VectorSubcoreMesh(core_axis_name="c", subcore_axis_name="s")
    gather = pl.kernel(
        functools.partial(_gather_body, spans_per_b=spans_per_b),
        mesh=mesh,
        out_type=jax.ShapeDtypeStruct((b, _DEPTH, t), jnp.float32),
        scratch_types=[
            pltpu.VMEM((_SPAN,), jnp.int32),
            pltpu.VMEM((_DEPTH, 256), jnp.float32),
            pltpu.VMEM((_DEPTH, _SPAN), jnp.float32),
            pltpu.SemaphoreType.DMA,
        ],
        compiler_params=pltpu.CompilerParams(
            use_tc_tiling_on_sc=True,
            needs_layout_passes=False,
            disable_bounds_checks=True,
            disable_semaphore_checks=True,
            skip_device_barrier=True,
        ),
    )
    out_t = gather(tablet, x)
    return jnp.transpose(out_t, (0, 2, 1))
